# baseline (speedup 1.0000x reference)
import functools

import jax
import jax.numpy as jnp
from jax import lax
from jax.experimental import pallas as pl
from jax.experimental.pallas import tpu as pltpu
from jax.experimental.pallas import tpu_sc as plsc

EMBED = 64
NW = 32
CHUNK = 128


@functools.cache
def _make_gather(total: int):
    per_w = total // NW
    nchunk = per_w // CHUNK
    mesh = plsc.VectorSubcoreMesh(core_axis_name="c", subcore_axis_name="s")

    @functools.partial(
        pl.kernel,
        mesh=mesh,
        out_type=jax.ShapeDtypeStruct((total, EMBED), jnp.float32),
        scratch_types=[
            pltpu.VMEM((nchunk, CHUNK), jnp.int32),
            pltpu.VMEM((CHUNK, EMBED), jnp.float32),
            pltpu.SemaphoreType.DMA,
        ],
        compiler_params=pltpu.CompilerParams(use_tc_tiling_on_sc=False),
    )
    def gather(idx_hbm, table_hbm, out_hbm, idx_v, rows_v, gsem):
        wid = lax.axis_index("s") * 2 + lax.axis_index("c")
        base = wid * per_w
        pltpu.sync_copy(idx_hbm.at[wid], idx_v)

        def body(j, carry):
            pltpu.async_copy(table_hbm.at[idx_v.at[j]], rows_v, gsem).wait()
            pltpu.sync_copy(rows_v, out_hbm.at[pl.ds(base + j * CHUNK, CHUNK)])
            return carry

        lax.fori_loop(0, nchunk, body, 0)

    return gather


def kernel(x, word_embed):
    total = x.size
    idx = x.astype(jnp.int32).reshape(NW, total // (NW * CHUNK), CHUNK)
    out = _make_gather(total)(idx, word_embed)
    return out.reshape(x.shape + (EMBED,))


# speedup vs baseline: 1.0367x; 1.0367x over previous
import functools

import jax
import jax.numpy as jnp
from jax import lax
from jax.experimental import pallas as pl
from jax.experimental.pallas import tpu as pltpu
from jax.experimental.pallas import tpu_sc as plsc

EMBED = 64
NW = 32
CHUNK = 128
K = 5


@functools.cache
def _make_gather(total: int):
    per_w = total // NW
    nchunk = per_w // CHUNK
    ngroups = nchunk // K
    assert nchunk % K == 0 and ngroups % 2 == 0
    mesh = plsc.VectorSubcoreMesh(core_axis_name="c", subcore_axis_name="s")

    @functools.partial(
        pl.kernel,
        mesh=mesh,
        out_type=jax.ShapeDtypeStruct((total, EMBED), jnp.float32),
        scratch_types=[
            pltpu.VMEM((nchunk, CHUNK), jnp.int32),
            pltpu.VMEM((2, K, CHUNK, EMBED), jnp.float32),
            pltpu.SemaphoreType.DMA,
            pltpu.SemaphoreType.DMA,
        ],
        compiler_params=pltpu.CompilerParams(use_tc_tiling_on_sc=False),
    )
    def gather(idx_hbm, table_hbm, out_hbm, idx_v, rows_v, gsem, osem):
        wid = lax.axis_index("s") * 2 + lax.axis_index("c")
        base = wid * per_w
        pltpu.sync_copy(idx_hbm.at[wid], idx_v)

        def fire_gathers(g, half):
            for b in range(K):
                pltpu.async_copy(
                    table_hbm.at[idx_v.at[g * K + b]], rows_v.at[half, b], gsem)

        def wait_gathers(g, half):
            for b in range(K):
                pltpu.make_async_copy(
                    table_hbm.at[idx_v.at[g * K + b]], rows_v.at[half, b], gsem).wait()

        def fire_outcopies(g, half):
            for b in range(K):
                pltpu.async_copy(
                    rows_v.at[half, b],
                    out_hbm.at[pl.ds(base + (g * K + b) * CHUNK, CHUNK)], osem)

        def wait_outcopies(half):
            for b in range(K):
                pltpu.make_async_copy(
                    rows_v.at[half, b], out_hbm.at[pl.ds(base, CHUNK)], osem).wait()

        fire_gathers(0, 0)

        def body(g, carry):
            wait_gathers(g, 0)

            @pl.when(g > 0)
            def _():
                wait_outcopies(1)

            fire_gathers(g + 1, 1)
            fire_outcopies(g, 0)
            wait_gathers(g + 1, 1)
            wait_outcopies(0)

            @pl.when(g + 2 < ngroups)
            def _():
                fire_gathers(g + 2, 0)

            fire_outcopies(g + 1, 1)
            return carry

        lax.fori_loop(0, ngroups // 2, lambda i, c: body(i * 2, c), 0, unroll=False)
        wait_outcopies(1)

    return gather


def kernel(x, word_embed):
    total = x.size
    idx = x.astype(jnp.int32).reshape(NW, total // (NW * CHUNK), CHUNK)
    out = _make_gather(total)(idx, word_embed)
    return out.reshape(x.shape + (EMBED,))


# speedup vs baseline: 1.0401x; 1.0033x over previous
import functools

import jax
import jax.numpy as jnp
from jax import lax
from jax.experimental import pallas as pl
from jax.experimental.pallas import tpu as pltpu
from jax.experimental.pallas import tpu_sc as plsc

EMBED = 64
NW = 32
CHUNK = 128
K = 5


@functools.cache
def _make_gather(batch: int, hist: int, vocab: int):
    assert batch % (NW * CHUNK) == 0
    nchunk = hist
    ngroups = nchunk // K
    assert nchunk % K == 0 and ngroups % 2 == 0
    mesh = plsc.VectorSubcoreMesh(core_axis_name="c", subcore_axis_name="s")

    @functools.partial(
        pl.kernel,
        mesh=mesh,
        out_type=jax.ShapeDtypeStruct((batch, hist, EMBED), jnp.float32),
        scratch_types=[
            pltpu.VMEM((nchunk, CHUNK), jnp.int32),
            pltpu.VMEM((2, K, CHUNK, EMBED), jnp.float32),
            pltpu.SemaphoreType.DMA,
            pltpu.SemaphoreType.DMA,
        ],
        compiler_params=pltpu.CompilerParams(use_tc_tiling_on_sc=False),
    )
    def gather(xt_hbm, table_hbm, out_hbm, idx_v, rows_v, gsem, osem):
        wid = lax.axis_index("s") * 2 + lax.axis_index("c")
        b0 = wid * CHUNK
        pltpu.sync_copy(xt_hbm.at[:, pl.ds(b0, CHUNK)], idx_v)

        def fire_gathers(g, half):
            for b in range(K):
                pltpu.async_copy(
                    table_hbm.at[idx_v.at[g * K + b]], rows_v.at[half, b], gsem)

        def wait_gathers(g, half):
            for b in range(K):
                pltpu.make_async_copy(
                    table_hbm.at[idx_v.at[g * K + b]], rows_v.at[half, b], gsem).wait()

        def fire_outcopies(g, half):
            for b in range(K):
                pltpu.async_copy(
                    rows_v.at[half, b],
                    out_hbm.at[pl.ds(b0, CHUNK), g * K + b], osem)

        def wait_outcopies(half):
            for b in range(K):
                pltpu.make_async_copy(
                    rows_v.at[half, b], out_hbm.at[pl.ds(b0, CHUNK), 0], osem).wait()

        fire_gathers(0, 0)

        def body(g, carry):
            wait_gathers(g, 0)

            @pl.when(g > 0)
            def _():
                wait_outcopies(1)

            fire_gathers(g + 1, 1)
            fire_outcopies(g, 0)
            wait_gathers(g + 1, 1)
            wait_outcopies(0)

            @pl.when(g + 2 < ngroups)
            def _():
                fire_gathers(g + 2, 0)

            fire_outcopies(g + 1, 1)
            return carry

        lax.fori_loop(0, ngroups // 2, lambda i, c: body(i * 2, c), 0, unroll=False)
        wait_outcopies(1)

    return gather


def kernel(x, word_embed):
    batch, hist = x.shape
    xt = x.astype(jnp.int32).T
    return _make_gather(batch, hist, word_embed.shape[0])(xt, word_embed)
